# Initial kernel scaffold; baseline (speedup 1.0000x reference)
import jax
import jax.numpy as jnp
from jax.experimental import pallas as pl


def kernel(word_ids, sememe_ids, embedding):
    raise NotImplementedError("write your pallas kernel here")


# speedup vs baseline: 6.1555x; 6.1555x over previous
import functools

import jax
import jax.numpy as jnp
from jax import lax
from jax.experimental import pallas as pl
from jax.experimental.pallas import tpu as pltpu
from jax.experimental.pallas import tpu_sc as plsc

_H = 128
_B = 4096
_W = 10
_S = 50
_EPS = 1e-6
_P = _B * _W
_NW = 32
_PPW = _P // _NW
_NB = 4
_NBLK = _PPW // _NB
_BPW = _B // _NW
_GROUP = 104
_NEG = float(-3.0e38)

_mesh = plsc.VectorSubcoreMesh(core_axis_name="c", subcore_axis_name="s")


_scratch_types = [
    pltpu.VMEM((2, _GROUP), jnp.int32),
    pltpu.VMEM((2, _GROUP), jnp.int32),
    pltpu.VMEM((2, _GROUP, _H), jnp.float32),
    pltpu.VMEM((2, _GROUP, _H), jnp.float32),
    pltpu.VMEM((64,), jnp.float32),
    pltpu.VMEM((64,), jnp.int32),
    pltpu.VMEM((16,), jnp.int32),
    pltpu.VMEM((_BPW, _H), jnp.float32),
    pltpu.SemaphoreType.DMA,
    pltpu.SemaphoreType.DMA,
]


def _sememe_knn_body(table_hbm, ids_hbm, out_hbm,
                idxA, idxB, rowsA, rowsB, skey, sval, fvb, oacc,
                semA, semB):
    wid = lax.axis_index("s") * 2 + lax.axis_index("c")
    blk_base = wid * _NBLK
    lane = lax.broadcasted_iota(jnp.int32, (16,), 0)
    zeros16 = jnp.zeros((16,), jnp.float32)
    neg16 = jnp.full((16,), _NEG, jnp.float32)

    def zero_init(i, carry):
        for c in range(8):
            oacc[i, pl.ds(c * 16, 16)] = zeros16
        return carry

    lax.fori_loop(0, _BPW, zero_init, 0)

    def load_idx(blk, idxv):
        pltpu.sync_copy(ids_hbm.at[blk], idxv)

    def fire(idxv, rowsv, sem):
        for g2 in range(2):
            pltpu.async_copy(table_hbm.at[idxv.at[g2]], rowsv.at[g2], sem)

    def drain(idxv, rowsv, sem):
        for g2 in range(2):
            pltpu.make_async_copy(
                table_hbm.at[idxv.at[g2]], rowsv.at[g2], sem).wait()

    def compute_pair(rowsv, blk_local, pi):
        gi, oi = divmod(pi, 2)
        s_off = oi * 51
        sp = [rowsv[gi, s_off, pl.ds(c * 16, 16)] + _EPS for c in range(8)]

        def sqdist_rows(base_row, r):
            acc = None
            for c in range(8):
                d = sp[c] - rowsv[gi, base_row + r, pl.ds(c * 16, 16)]
                acc = d * d if acc is None else acc + d * d
            return jnp.sum(acc)

        base = s_off + 1
        for g in range(4):
            if g < 3:
                def group_body(it, tot, g=g):
                    r = it * 2
                    s0 = sqdist_rows(base + g * 16, r)
                    s1 = sqdist_rows(base + g * 16, r + 1)
                    tot = jnp.where(lane == r, s0, tot)
                    return jnp.where(lane == r + 1, s1, tot)

                tot = lax.fori_loop(0, 8, group_body, neg16)
            else:
                s0 = sqdist_rows(base + 48, 0)
                s1 = sqdist_rows(base + 48, 1)
                tot = jnp.where(lane == 0, s0, neg16)
                tot = jnp.where(lane == 1, s1, tot)
            sk, sv = plsc.sort_key_val(tot, lane + g * 16, descending=True)
            skey[pl.ds(g * 16, 16)] = sk
            sval[pl.ds(g * 16, 16)] = sv

        cidx = jnp.where(lane < 12, (lane // 3) * 16 + lane % 3, 0)
        ck = plsc.load_gather(skey, [cidx])
        cv = plsc.load_gather(sval, [cidx])
        ck = jnp.where(lane < 12, ck, _NEG)
        _, fv = plsc.sort_key_val(ck, cv, descending=False)
        fvb[...] = fv
        j0 = plsc.load_gather(fvb, [lane * 0 + 15])
        j1 = plsc.load_gather(fvb, [lane * 0 + 14])
        j2 = plsc.load_gather(fvb, [lane * 0 + 13])

        rows_g = rowsv.at[gi]
        r0 = base + j0
        r1 = base + j1
        r2 = base + j2
        pair_local = blk_local * _NB + pi
        lb = pair_local // _W
        for c in range(8):
            col = lane + c * 16
            e0 = plsc.load_gather(rows_g, [r0, col])
            e1 = plsc.load_gather(rows_g, [r1, col])
            e2 = plsc.load_gather(rows_g, [r2, col])
            s_c = rowsv[gi, s_off, pl.ds(c * 16, 16)]
            contrib = (s_c + (e0 + e1 + e2) * (1.0 / 3.0)) * (1.0 / (2 * _W))
            oacc[lb, pl.ds(c * 16, 16)] = oacc[lb, pl.ds(c * 16, 16)] + contrib

    def compute_block(rowsv, blk_local):
        for pi in range(_NB):
            compute_pair(rowsv, blk_local, pi)

    load_idx(blk_base, idxA)
    fire(idxA, rowsA, semA)
    load_idx(blk_base + 1, idxB)
    fire(idxB, rowsB, semB)

    def body2(kk, carry):
        b0 = 2 * kk
        drain(idxA, rowsA, semA)
        compute_block(rowsA, b0)

        @pl.when(b0 + 2 < _NBLK)
        def _():
            load_idx(blk_base + b0 + 2, idxA)
            fire(idxA, rowsA, semA)

        drain(idxB, rowsB, semB)
        compute_block(rowsB, b0 + 1)

        @pl.when(b0 + 3 < _NBLK)
        def _():
            load_idx(blk_base + b0 + 3, idxB)
            fire(idxB, rowsB, semB)

        return carry

    lax.fori_loop(0, _NBLK // 2, body2, 0)
    pltpu.sync_copy(oacc, out_hbm.at[pl.ds(wid * _BPW, _BPW)])


_sememe_knn = pl.kernel(
    _sememe_knn_body,
    out_type=jax.ShapeDtypeStruct((_B, _H), jnp.float32),
    mesh=_mesh,
    compiler_params=pltpu.CompilerParams(needs_layout_passes=False),
    scratch_types=_scratch_types,
)


def kernel(word_ids, sememe_ids, embedding):
    word_ids = word_ids.astype(jnp.int32)
    sememe_ids = sememe_ids.astype(jnp.int32)
    ids = jnp.concatenate([word_ids[:, :, None], sememe_ids], axis=2)
    ids = ids.reshape(_P // 2, 2 * (_S + 1))
    ids = jnp.pad(ids, ((0, 0), (0, _GROUP - 2 * (_S + 1))))
    ids = ids.reshape(_P // _NB, 2, _GROUP)
    return _sememe_knn(embedding, ids)
